# Initial kernel scaffold; baseline (speedup 1.0000x reference)
import functools

import jax
import jax.numpy as jnp
from jax import lax
from jax.experimental import pallas as pl
from jax.experimental.pallas import tpu as pltpu
from jax.experimental.pallas import tpu_sc as plsc

_EMBED = 64
_GATHER = 128
_K = 4
_GROUP = _GATHER * _K


@functools.lru_cache(maxsize=None)
def _make_kernel(B: int):
  info = plsc.get_sparse_core_info()
  nw = info.num_cores * info.num_subcores
  b_per_w = B // nw
  n_groups = b_per_w // _GROUP
  idx_rows = b_per_w // _GATHER

  mesh = plsc.VectorSubcoreMesh(core_axis_name="c", subcore_axis_name="s")

  @functools.partial(
      pl.kernel,
      mesh=mesh,
      out_type=jax.ShapeDtypeStruct((B, _EMBED), jnp.float32),
      scratch_types=[
          pltpu.VMEM((idx_rows, _GATHER), jnp.int32),
          pltpu.VMEM((_GROUP, _EMBED), jnp.float32),
          pltpu.SemaphoreType.DMA,
      ],
  )
  def k(idx_hbm, table_hbm, out_hbm, idx_v, rows_v, sem):
    wid = lax.axis_index("s") * info.num_cores + lax.axis_index("c")
    base = wid * b_per_w
    pltpu.sync_copy(idx_hbm.at[pl.ds(wid * idx_rows, idx_rows)], idx_v)

    @pl.loop(0, n_groups)
    def _(g):
      for j in range(_K):
        pltpu.async_copy(
            table_hbm.at[idx_v.at[g * _K + j]],
            rows_v.at[pl.ds(j * _GATHER, _GATHER)],
            sem,
        )
      for j in range(_K):
        pltpu.make_async_copy(
            table_hbm.at[idx_v.at[g * _K + j]],
            rows_v.at[pl.ds(j * _GATHER, _GATHER)],
            sem,
        ).wait()
      pltpu.sync_copy(rows_v, out_hbm.at[pl.ds(base + g * _GROUP, _GROUP)])

  return k


def kernel(token, table):
  B = token.shape[0] * token.shape[1]
  idx2d = token.reshape(B // _GATHER, _GATHER)
  out = _make_kernel(B)(idx2d, table)
  return out.reshape(token.shape[0], token.shape[1], _EMBED)


# speedup vs baseline: 1.8314x; 1.8314x over previous
import functools

import jax
import jax.numpy as jnp
from jax import lax
from jax.experimental import pallas as pl
from jax.experimental.pallas import tpu as pltpu
from jax.experimental.pallas import tpu_sc as plsc

_EMBED = 64
_GATHER = 128
_K = 4
_GROUP = _GATHER * _K


@functools.lru_cache(maxsize=None)
def _make_kernel(B: int):
  info = plsc.get_sparse_core_info()
  nw = info.num_cores * info.num_subcores
  b_per_w = B // nw
  n_groups = b_per_w // _GROUP
  idx_rows = b_per_w // _GATHER

  mesh = plsc.VectorSubcoreMesh(core_axis_name="c", subcore_axis_name="s")

  @functools.partial(
      pl.kernel,
      mesh=mesh,
      out_type=jax.ShapeDtypeStruct((B, _EMBED), jnp.float32),
      scratch_types=[
          pltpu.VMEM((idx_rows, _GATHER), jnp.int32),
          pltpu.VMEM((_GROUP, _EMBED), jnp.float32),
          pltpu.SemaphoreType.DMA,
      ],
      compiler_params=pltpu.CompilerParams(use_tc_tiling_on_sc=False),
  )
  def k(idx_hbm, table_hbm, out_hbm, idx_v, rows_v, sem):
    wid = lax.axis_index("s") * info.num_cores + lax.axis_index("c")
    base = wid * b_per_w
    pltpu.sync_copy(idx_hbm.at[pl.ds(wid * idx_rows, idx_rows)], idx_v)

    @pl.loop(0, n_groups)
    def _(g):
      for j in range(_K):
        pltpu.async_copy(
            table_hbm.at[idx_v.at[g * _K + j]],
            rows_v.at[pl.ds(j * _GATHER, _GATHER)],
            sem,
        )
      for j in range(_K):
        pltpu.make_async_copy(
            table_hbm.at[idx_v.at[g * _K + j]],
            rows_v.at[pl.ds(j * _GATHER, _GATHER)],
            sem,
        ).wait()
      pltpu.sync_copy(rows_v, out_hbm.at[pl.ds(base + g * _GROUP, _GROUP)])

  return k


def kernel(token, table):
  B = token.shape[0] * token.shape[1]
  idx2d = token.reshape(B // _GATHER, _GATHER)
  out = _make_kernel(B)(idx2d, table)
  return out.reshape(token.shape[0], token.shape[1], _EMBED)


# speedup vs baseline: 1.8757x; 1.0242x over previous
import functools

import jax
import jax.numpy as jnp
from jax import lax
from jax.experimental import pallas as pl
from jax.experimental.pallas import tpu as pltpu
from jax.experimental.pallas import tpu_sc as plsc

_EMBED = 64
_GATHER = 128
_K = 4
_GROUP = _GATHER * _K


@functools.lru_cache(maxsize=None)
def _make_kernel(B: int):
  info = plsc.get_sparse_core_info()
  nw = info.num_cores * info.num_subcores
  b_per_w = B // nw
  n_groups = b_per_w // _GROUP
  idx_rows = b_per_w // _GATHER

  mesh = plsc.VectorSubcoreMesh(core_axis_name="c", subcore_axis_name="s")

  @functools.partial(
      pl.kernel,
      mesh=mesh,
      out_type=jax.ShapeDtypeStruct((B, _EMBED), jnp.float32),
      scratch_types=[
          pltpu.VMEM((idx_rows, _GATHER), jnp.int32),
          pltpu.VMEM((2, _GROUP, _EMBED), jnp.float32),
          pltpu.SemaphoreType.DMA,
          pltpu.SemaphoreType.DMA,
      ],
      compiler_params=pltpu.CompilerParams(use_tc_tiling_on_sc=False),
  )
  def k(idx_hbm, table_hbm, out_hbm, idx_v, rows_v, gat_sem, out_sem):
    wid = lax.axis_index("s") * info.num_cores + lax.axis_index("c")
    base = wid * b_per_w
    pltpu.sync_copy(idx_hbm.at[pl.ds(wid * idx_rows, idx_rows)], idx_v)

    def fire_gathers(g, slot):
      for j in range(_K):
        pltpu.async_copy(
            table_hbm.at[idx_v.at[g * _K + j]],
            rows_v.at[slot, pl.ds(j * _GATHER, _GATHER)],
            gat_sem,
        )

    def wait_gathers(g, slot):
      for j in range(_K):
        pltpu.make_async_copy(
            table_hbm.at[idx_v.at[g * _K + j]],
            rows_v.at[slot, pl.ds(j * _GATHER, _GATHER)],
            gat_sem,
        ).wait()

    def out_desc(g, slot):
      return pltpu.make_async_copy(
          rows_v.at[slot],
          out_hbm.at[pl.ds(base + g * _GROUP, _GROUP)],
          out_sem,
      )

    fire_gathers(0, 0)

    @pl.loop(0, n_groups)
    def _(g):
      slot = lax.rem(g, 2)
      nxt = 1 - slot

      @pl.when(g >= 1)
      def _():
        out_desc(g - 1, nxt).wait()

      @pl.when(g + 1 < n_groups)
      def _():
        fire_gathers(g + 1, nxt)

      wait_gathers(g, slot)
      out_desc(g, slot).start()

    out_desc(n_groups - 1, lax.rem(n_groups - 1, 2)).wait()

  return k


def kernel(token, table):
  B = token.shape[0] * token.shape[1]
  idx2d = token.reshape(B // _GATHER, _GATHER)
  out = _make_kernel(B)(idx2d, table)
  return out.reshape(token.shape[0], token.shape[1], _EMBED)


# speedup vs baseline: 2.3275x; 1.2409x over previous
import functools

import jax
import jax.numpy as jnp
from jax import lax
from jax.experimental import pallas as pl
from jax.experimental.pallas import tpu as pltpu
from jax.experimental.pallas import tpu_sc as plsc

_EMBED = 64
_BLK = 128


@functools.lru_cache(maxsize=None)
def _make_kernel(J: int, I: int, V: int):
  info = plsc.get_sparse_core_info()
  nc = info.num_cores
  nw = nc * info.num_subcores
  n_blocks = J * (I // _BLK)
  bpw = n_blocks // nw
  ib_count = I // _BLK

  mesh = plsc.VectorSubcoreMesh(core_axis_name="c", subcore_axis_name="s")

  @functools.partial(
      pl.kernel,
      mesh=mesh,
      out_type=jax.ShapeDtypeStruct((J, 8, ib_count, 8, _BLK), jnp.float32),
      scratch_types=[
          pltpu.VMEM((bpw, _BLK), jnp.int32),
          pltpu.VMEM((_BLK, _EMBED), jnp.float32),
          pltpu.VMEM((_BLK, _EMBED), jnp.float32),
          pltpu.VMEM((8, 8, _BLK + 1), jnp.float32),
          pltpu.VMEM((8, 8, _BLK + 1), jnp.float32),
          pltpu.SemaphoreType.DMA,
          pltpu.SemaphoreType.DMA,
      ],
      compiler_params=pltpu.CompilerParams(
          use_tc_tiling_on_sc=False, needs_layout_passes=False
      ),
  )
  def k(tok_hbm, table_hbm, out_hbm, idx_v, g0, g1, t0, t1, gsem, osem):
    wid = lax.axis_index("s") * nc + lax.axis_index("c")
    base = wid * bpw
    pltpu.sync_copy(tok_hbm.at[pl.ds(base, bpw)], idx_v)

    kv = lax.iota(jnp.int32, 16)
    clv = lax.rem(kv, 8)
    chv = [c0 * 2 + lax.div(kv, 8) for c0 in range(4)]

    def fire_gather(b, g):
      pltpu.async_copy(table_hbm.at[idx_v.at[b]], g, gsem)

    def wait_gather(b, g):
      pltpu.make_async_copy(table_hbm.at[idx_v.at[b]], g, gsem).wait()

    def out_desc(b, t):
      gb = base + b
      j = lax.div(gb, ib_count)
      ib = lax.rem(gb, ib_count)
      return pltpu.make_async_copy(
          t.at[:, :, pl.ds(0, _BLK)],
          out_hbm.at[j, :, ib],
          osem,
      )

    def transpose(g, t):
      @pl.loop(0, _BLK)
      def _(il):
        ilv = jnp.full((16,), il, jnp.int32)
        for c0 in range(4):
          x = g[il, pl.ds(c0 * 16, 16)]
          plsc.store_scatter(t, [chv[c0], clv, ilv], x)

    fire_gather(0, g0)

    @pl.loop(0, bpw // 2)
    def _(p):
      b0 = p * 2
      b1 = b0 + 1

      @pl.when(p > 0)
      def _():
        out_desc(b0 - 2, t0).wait()

      fire_gather(b1, g1)
      wait_gather(b0, g0)
      transpose(g0, t0)
      out_desc(b0, t0).start()

      @pl.when(p > 0)
      def _():
        out_desc(b1 - 2, t1).wait()

      @pl.when(p + 1 < bpw // 2)
      def _():
        fire_gather(b0 + 2, g0)

      wait_gather(b1, g1)
      transpose(g1, t1)
      out_desc(b1, t1).start()

    out_desc(bpw - 2, t0).wait()
    out_desc(bpw - 1, t1).wait()

  return k


def kernel(token, table):
  I, J = token.shape
  V = table.shape[0]
  tok = token.T.reshape(J * I // _BLK, _BLK)
  out5 = _make_kernel(J, I, V)(tok, table)
  return out5.transpose(2, 4, 0, 1, 3).reshape(I, J, _EMBED)


# speedup vs baseline: 2.4084x; 1.0347x over previous
import functools

import jax
import jax.numpy as jnp
from jax import lax
from jax.experimental import pallas as pl
from jax.experimental.pallas import tpu as pltpu
from jax.experimental.pallas import tpu_sc as plsc

_EMBED = 64
_BLK = 128


@functools.lru_cache(maxsize=None)
def _make_kernel(J: int, I: int, V: int):
  info = plsc.get_sparse_core_info()
  nc = info.num_cores
  nw = nc * info.num_subcores
  n_blocks = J * (I // _BLK)
  bpw = n_blocks // nw
  ib_count = I // _BLK

  mesh = plsc.VectorSubcoreMesh(core_axis_name="c", subcore_axis_name="s")

  @functools.partial(
      pl.kernel,
      mesh=mesh,
      out_type=jax.ShapeDtypeStruct((J, 8, ib_count, 8, _BLK), jnp.float32),
      scratch_types=[
          pltpu.VMEM((bpw, _BLK), jnp.int32),
          pltpu.VMEM((_BLK, _EMBED), jnp.float32),
          pltpu.VMEM((_BLK, _EMBED), jnp.float32),
          pltpu.VMEM((8, 8, _BLK + 1), jnp.float32),
          pltpu.VMEM((8, 8, _BLK + 1), jnp.float32),
          pltpu.SemaphoreType.DMA,
          pltpu.SemaphoreType.DMA,
      ],
      compiler_params=pltpu.CompilerParams(
          use_tc_tiling_on_sc=False, needs_layout_passes=False
      ),
  )
  def k(tok_hbm, table_hbm, out_hbm, idx_v, g0, g1, t0, t1, gsem, osem):
    wid = lax.axis_index("s") * nc + lax.axis_index("c")
    base = wid * bpw
    pltpu.sync_copy(tok_hbm.at[pl.ds(base, bpw)], idx_v)

    kv = lax.iota(jnp.int32, 16)
    clv = lax.rem(kv, 8)
    chv = [c0 * 2 + lax.div(kv, 8) for c0 in range(4)]

    def fire_gather(b, g):
      pltpu.async_copy(table_hbm.at[idx_v.at[b]], g, gsem)

    def wait_gather(b, g):
      pltpu.make_async_copy(table_hbm.at[idx_v.at[b]], g, gsem).wait()

    def out_desc(b, t):
      gb = base + b
      j = lax.div(gb, ib_count)
      ib = lax.rem(gb, ib_count)
      return pltpu.make_async_copy(
          t.at[:, :, pl.ds(0, _BLK)],
          out_hbm.at[j, :, ib],
          osem,
      )

    def transpose(g, t):
      @pl.loop(0, _BLK, unroll=8)
      def _(il):
        ilv = jnp.full((16,), il, jnp.int32)
        for c0 in range(4):
          x = g[il, pl.ds(c0 * 16, 16)]
          plsc.store_scatter(t, [chv[c0], clv, ilv], x)

    fire_gather(0, g0)

    @pl.loop(0, bpw // 2)
    def _(p):
      b0 = p * 2
      b1 = b0 + 1

      @pl.when(p > 0)
      def _():
        out_desc(b0 - 2, t0).wait()

      fire_gather(b1, g1)
      wait_gather(b0, g0)
      transpose(g0, t0)
      out_desc(b0, t0).start()

      @pl.when(p > 0)
      def _():
        out_desc(b1 - 2, t1).wait()

      @pl.when(p + 1 < bpw // 2)
      def _():
        fire_gather(b0 + 2, g0)

      wait_gather(b1, g1)
      transpose(g1, t1)
      out_desc(b1, t1).start()

    out_desc(bpw - 2, t0).wait()
    out_desc(bpw - 1, t1).wait()

  return k


def kernel(token, table):
  I, J = token.shape
  V = table.shape[0]
  tok = token.T.reshape(J * I // _BLK, _BLK)
  out5 = _make_kernel(J, I, V)(tok, table)
  return out5.transpose(2, 4, 0, 1, 3).reshape(I, J, _EMBED)
